# baseline (speedup 1.0000x reference)
import functools

import jax
import jax.numpy as jnp
from jax import lax
from jax.experimental import pallas as pl
from jax.experimental.pallas import tpu as pltpu
from jax.experimental.pallas import tpu_sc as plsc

_SEQ_LENS = (2048, 1792, 1536, 1280, 1024, 768, 512, 256)
_D = 1024
_MAXL = 2048
_NC = 2
_ZROWS = 64


def _pad_body(x0, x1, x2, x3, x4, x5, x6, x7, zsrc, out, zbuf, csem, zwsem):
    xs = (x0, x1, x2, x3, x4, x5, x6, x7)
    w = lax.axis_index("s") * _NC + lax.axis_index("c")
    pltpu.sync_copy(zsrc, zbuf)
    for seq in range(8):
        L = _SEQ_LENS[seq]
        for q in range(4):
            c = min(max(L - 512 * q, 0), 512)
            nz = (512 - c) // _ZROWS
            base = seq * _MAXL + q * 512

            @pl.when(w == seq * 4 + q)
            def _(seq=seq, q=q, c=c, nz=nz, base=base):
                for k in range(nz):
                    pltpu.async_copy(
                        zbuf, out.at[pl.ds(base + c + k * _ZROWS, _ZROWS), :],
                        zwsem)
                if c > 0:
                    pltpu.async_copy(
                        xs[seq].at[pl.ds(q * 512, c), :],
                        out.at[pl.ds(base, c), :], csem)
                    pltpu.make_async_copy(
                        xs[seq].at[pl.ds(q * 512, c), :],
                        out.at[pl.ds(base, c), :], csem).wait()
                for k in range(nz):
                    pltpu.make_async_copy(
                        zbuf, out.at[pl.ds(base + c, _ZROWS), :], zwsem).wait()


@functools.partial(
    pl.kernel,
    out_type=jax.ShapeDtypeStruct((8 * _MAXL, _D), jnp.float32),
    mesh=plsc.VectorSubcoreMesh(core_axis_name="c", subcore_axis_name="s"),
    scratch_types=[
        pltpu.VMEM((_ZROWS, _D), jnp.float32),
        pltpu.SemaphoreType.DMA,
        pltpu.SemaphoreType.DMA,
    ],
)
def _pad_sc(*refs):
    _pad_body(*refs)


def kernel(x0, x1, x2, x3, x4, x5, x6, x7):
    zsrc = jnp.zeros((_ZROWS, _D), jnp.float32)
    out = _pad_sc(x0, x1, x2, x3, x4, x5, x6, x7, zsrc)
    return out.reshape(8, _MAXL, _D)


# speedup vs baseline: 15.7761x; 15.7761x over previous
import functools

import jax
import jax.numpy as jnp
from jax import lax
from jax.experimental import pallas as pl
from jax.experimental.pallas import tpu as pltpu
from jax.experimental.pallas import tpu_sc as plsc

_SEQ_LENS = (2048, 1792, 1536, 1280, 1024, 768, 512, 256)
_D = 1024
_MAXL = 2048
_NC = 2
_CH = 32
_NB = 3
_ZROWS = 16


def _copy_quarter(x, out, r0, base, c, bufs, rsems, wsems):
    n = c // _CH

    def rd(k, b):
        pltpu.async_copy(x.at[pl.ds(r0 + k * _CH, _CH), :], bufs[b],
                         rsems.at[b])

    def wr(k, b):
        pltpu.async_copy(bufs[b], out.at[pl.ds(base + k * _CH, _CH), :],
                         wsems.at[b])

    for k in range(min(_NB - 1, n)):
        rd(k, k)
    for k in range(n):
        b = k % _NB
        pltpu.make_async_copy(x.at[pl.ds(r0 + k * _CH, _CH), :], bufs[b],
                              rsems.at[b]).wait()
        wr(k, b)
        nxt = k + _NB - 1
        if nxt < n:
            bn = nxt % _NB
            if k > 0:
                pltpu.make_async_copy(
                    bufs[bn], out.at[pl.ds(base + (k - 1) * _CH, _CH), :],
                    wsems.at[bn]).wait()
            rd(nxt, bn)
    for k in range(max(0, n - _NB), n):
        b = k % _NB
        pltpu.make_async_copy(bufs[b], out.at[pl.ds(base + k * _CH, _CH), :],
                              wsems.at[b]).wait()


def _pad_body(x0, x1, x2, x3, x4, x5, x6, x7, zsrc, out, zbuf, buf0, buf1,
              buf2, rsems, wsems, zwsem):
    xs = (x0, x1, x2, x3, x4, x5, x6, x7)
    bufs = (buf0, buf1, buf2)
    w = lax.axis_index("s") * _NC + lax.axis_index("c")
    pltpu.sync_copy(zsrc, zbuf)
    for seq in range(8):
        L = _SEQ_LENS[seq]
        for q in range(4):
            c = min(max(L - 512 * q, 0), 512)
            nz = (512 - c) // _ZROWS
            base = seq * _MAXL + q * 512

            @pl.when(w == seq * 4 + q)
            def _(seq=seq, q=q, c=c, nz=nz, base=base):
                for k in range(nz):
                    pltpu.async_copy(
                        zbuf, out.at[pl.ds(base + c + k * _ZROWS, _ZROWS), :],
                        zwsem)
                if c > 0:
                    _copy_quarter(xs[seq], out, q * 512, base, c, bufs, rsems,
                                  wsems)
                for k in range(nz):
                    pltpu.make_async_copy(
                        zbuf, out.at[pl.ds(base + c, _ZROWS), :], zwsem).wait()


@functools.partial(
    pl.kernel,
    out_type=jax.ShapeDtypeStruct((8 * _MAXL, _D), jnp.float32),
    mesh=plsc.VectorSubcoreMesh(core_axis_name="c", subcore_axis_name="s"),
    scratch_types=[
        pltpu.VMEM((_ZROWS, _D), jnp.float32),
        pltpu.VMEM((_CH, _D), jnp.float32),
        pltpu.VMEM((_CH, _D), jnp.float32),
        pltpu.VMEM((_CH, _D), jnp.float32),
        pltpu.SemaphoreType.DMA((_NB,)),
        pltpu.SemaphoreType.DMA((_NB,)),
        pltpu.SemaphoreType.DMA,
    ],
)
def _pad_sc(*refs):
    _pad_body(*refs)


def kernel(x0, x1, x2, x3, x4, x5, x6, x7):
    zsrc = jnp.zeros((_ZROWS, _D), jnp.float32)
    out = _pad_sc(x0, x1, x2, x3, x4, x5, x6, x7, zsrc)
    return out.reshape(8, _MAXL, _D)
